# baseline (speedup 1.0000x reference)
import functools

import numpy as np
import jax
import jax.numpy as jnp
from jax import lax
from jax.experimental import pallas as pl
from jax.experimental.pallas import tpu as pltpu
from jax.experimental.pallas import tpu_sc as plsc

POOLED = 7
SCALE = 1.0 / 16.0
CCHUNK = 32
NH_BUCKETS = (4, 8, 16, 26, 38)


def _round_half_even_v(x):
  r0 = x.astype(jnp.int32)
  frac = x - r0.astype(jnp.float32)
  gt = (frac > 0.5).astype(jnp.int32)
  eq = (frac == 0.5).astype(jnp.int32)
  return r0 + gt + eq * (r0 & 1)


def _floor_v(x):
  return x.astype(jnp.int32)


def _ceil_v(x):
  c = x.astype(jnp.int32)
  return c + (x > c.astype(jnp.float32)).astype(jnp.int32)


def _make_sc_call(N, C, B, H, W):
  Q = C // CCHUNK
  info = plsc.get_sparse_core_info()
  NC, NS = info.num_cores, info.num_subcores
  NW = NC * NS
  ipw = (N * Q) // NW
  qshift = Q.bit_length() - 1
  assert Q == 1 << qshift and ipw % 2 == 0

  def body(featq_hbm, rois_hbm, div7_hbm, out_hbm, roisv, div7v, slab0,
           slab1, ob0, ob1, semA, semB, semO0, semO1):
    wid = lax.axis_index("s") * NC + lax.axis_index("c")
    pltpu.sync_copy(rois_hbm, roisv)
    pltpu.sync_copy(div7_hbm, div7v)
    cidx = lax.iota(jnp.int32, 16)
    iot_f = cidx.astype(jnp.float32)

    def params(k):
      item = k * NW + wid
      n = lax.shift_right_logical(item, qshift)
      q = jnp.bitwise_and(item, Q - 1)
      rv = roisv[n]
      b = rv[0].astype(jnp.int32)
      rs = _round_half_even_v(rv * SCALE)
      rsw, rsh, rew, reh = rs[1], rs[2], rs[3], rs[4]
      roi_w = jnp.maximum(rew - rsw + 1, 1)
      roi_h = jnp.maximum(reh - rsh + 1, 1)
      bin_h = div7v[roi_h][0]
      bin_w = div7v[roi_w][0]
      hs_v = jnp.clip(_floor_v(iot_f * bin_h) + rsh, 0, H)
      he_v = jnp.clip(_ceil_v((iot_f + 1.0) * bin_h) + rsh, 0, H)
      ws_v = jnp.clip(_floor_v(iot_f * bin_w) + rsw, 0, W)
      we_v = jnp.clip(_ceil_v((iot_f + 1.0) * bin_w) + rsw, 0, W)
      h0 = hs_v[0]
      nh = he_v[POOLED - 1] - h0
      s_sel = jnp.int32(NH_BUCKETS[-1])
      for s in reversed(NH_BUCKETS[:-1]):
        s_sel = jnp.where(nh <= s, jnp.int32(s), s_sel)
      h0c = jnp.minimum(h0, H - s_sel)
      return dict(n=n, q=q, b=b, nh=nh, h0c=h0c,
                  hs=hs_v, he=he_v, ws=ws_v, we=we_v)

    def issue_in(prm, slab, sem):
      prev = 0
      for s in NH_BUCKETS:
        cond = (prm["nh"] > prev) & (prm["nh"] <= s) if prev else (prm["nh"] <= s)

        @pl.when(cond)
        def _(s=s):
          pltpu.async_copy(
              featq_hbm.at[prm["q"], prm["b"],
                           pl.ds(jnp.minimum(prm["h0c"], H - s), s)],
              slab.at[pl.ds(0, s)], sem)

        prev = s

    def drain_in(prm, slab, sem):
      prev = 0
      for s in NH_BUCKETS:
        cond = (prm["nh"] > prev) & (prm["nh"] <= s) if prev else (prm["nh"] <= s)

        @pl.when(cond)
        def _(s=s):
          pltpu.make_async_copy(
              featq_hbm.at[0, 0, pl.ds(0, s)], slab.at[pl.ds(0, s)], sem
          ).wait()

        prev = s

    def drain_out(ob, sem):
      pltpu.make_async_copy(out_hbm.at[0, pl.ds(0, CCHUNK)], ob, sem).wait()

    def compute(prm, slab, ob, semo, first):
      @pl.when(jnp.logical_not(first))
      def _():
        drain_out(ob, semo)

      h0c = prm["h0c"]
      neg = jnp.full((16,), -jnp.inf, jnp.float32)
      for ph in range(POOLED):
        hs, he = prm["hs"][ph], prm["he"][ph]
        phv = jnp.broadcast_to(jnp.int32(ph), (16,))
        for pw in range(POOLED):
          ws, we = prm["ws"][pw], prm["we"][pw]

          @plsc.parallel_loop(hs, he, unroll=1, carry=(neg, neg))
          def hloop(h, accs):
            row = h - h0c

            @plsc.parallel_loop(ws, we, unroll=4, carry=accs)
            def wloop(w, accs2):
              a0, a1 = accs2
              col = w * CCHUNK
              v0 = slab[row, pl.ds(col, 16)]
              v1 = slab[row, pl.ds(col + 16, 16)]
              return (jnp.maximum(a0, v0), jnp.maximum(a1, v1))

            return wloop

          a0, a1 = hloop
          emptyv = jnp.broadcast_to((he <= hs) | (we <= ws), (16,))
          r0 = jnp.where(emptyv, 0.0, a0)
          r1 = jnp.where(emptyv, 0.0, a1)
          pwv = jnp.broadcast_to(jnp.int32(pw), (16,))
          plsc.store_scatter(ob, [cidx, phv, pwv], r0)
          plsc.store_scatter(ob, [cidx + 16, phv, pwv], r1)
      pltpu.async_copy(
          ob, out_hbm.at[prm["n"], pl.ds(prm["q"] * CCHUNK, CCHUNK)], semo)

    p0 = params(0)
    issue_in(p0, slab0, semA)

    @pl.loop(0, ipw // 2)
    def _pair(p):
      k0 = 2 * p
      prm1 = params(k0 + 1)
      issue_in(prm1, slab1, semB)
      prm0 = params(k0)
      drain_in(prm0, slab0, semA)
      compute(prm0, slab0, ob0, semO0, first=(k0 == 0))
      prm2 = params(jnp.minimum(k0 + 2, ipw - 1))
      issue_in(prm2, slab0, semA)
      drain_in(prm1, slab1, semB)
      compute(prm1, slab1, ob1, semO1, first=(k0 == 0))

    plast = params(ipw - 1)
    drain_in(plast, slab0, semA)
    drain_out(ob0, semO0)
    drain_out(ob1, semO1)

  mesh = plsc.VectorSubcoreMesh(core_axis_name="c", subcore_axis_name="s")
  return pl.kernel(
      body,
      out_type=jax.ShapeDtypeStruct((N, C, POOLED, POOLED), jnp.float32),
      mesh=mesh,
      compiler_params=pltpu.CompilerParams(
          use_tc_tiling_on_sc=False, needs_layout_passes=False
      ),
      scratch_types=[
          pltpu.VMEM((N, 16), jnp.float32),
          pltpu.VMEM((64, 16), jnp.float32),
          pltpu.VMEM((H, W * CCHUNK), jnp.float32),
          pltpu.VMEM((H, W * CCHUNK), jnp.float32),
          pltpu.VMEM((CCHUNK, POOLED, POOLED), jnp.float32),
          pltpu.VMEM((CCHUNK, POOLED, POOLED), jnp.float32),
          pltpu.SemaphoreType.DMA,
          pltpu.SemaphoreType.DMA,
          pltpu.SemaphoreType.DMA,
          pltpu.SemaphoreType.DMA,
      ],
  )


@jax.jit
def kernel(features, rois):
  B, C, H, W = features.shape
  N = rois.shape[0]
  Q = C // CCHUNK
  featq = (
      features.reshape(B, Q, CCHUNK, H, W)
      .transpose(1, 0, 3, 4, 2)
      .reshape(Q, B, H, W * CCHUNK)
  )
  roisp = jnp.pad(rois, ((0, 0), (0, 16 - rois.shape[1])))
  div7 = jnp.asarray(
      np.broadcast_to(
          (
              np.arange(64, dtype=np.float32)
              * (np.float32(1.0) / np.float32(POOLED))
          )[:, None],
          (64, 16),
      )
  )
  return _make_sc_call(N, C, B, H, W)(featq, roisp, div7)


# speedup vs baseline: 2.0685x; 2.0685x over previous
import functools

import numpy as np
import jax
import jax.numpy as jnp
from jax import lax
from jax.experimental import pallas as pl
from jax.experimental.pallas import tpu as pltpu
from jax.experimental.pallas import tpu_sc as plsc

POOLED = 7
SCALE = 1.0 / 16.0
CCHUNK = 32
NH_BUCKETS = (4, 8, 16, 26, 38)


def _round_half_even_v(x):
  r0 = x.astype(jnp.int32)
  frac = x - r0.astype(jnp.float32)
  gt = (frac > 0.5).astype(jnp.int32)
  eq = (frac == 0.5).astype(jnp.int32)
  return r0 + gt + eq * (r0 & 1)


def _floor_v(x):
  return x.astype(jnp.int32)


def _ceil_v(x):
  c = x.astype(jnp.int32)
  return c + (x > c.astype(jnp.float32)).astype(jnp.int32)


def _make_sc_call(N, C, B, H, W):
  Q = C // CCHUNK
  info = plsc.get_sparse_core_info()
  NC, NS = info.num_cores, info.num_subcores
  NW = NC * NS
  ipw = (N * Q) // NW
  qshift = Q.bit_length() - 1
  assert Q == 1 << qshift and ipw % 2 == 0

  def body(featq_hbm, rois_hbm, div7_hbm, out_hbm, roisv, div7v, slab0,
           slab1, ob0, ob1, semA, semB, semO0, semO1):
    wid = lax.axis_index("s") * NC + lax.axis_index("c")
    pltpu.sync_copy(rois_hbm, roisv)
    pltpu.sync_copy(div7_hbm, div7v)
    cidx = lax.iota(jnp.int32, 16)
    iot_f = cidx.astype(jnp.float32)

    def params(k):
      item = k * NW + wid
      n = lax.shift_right_logical(item, qshift)
      q = jnp.bitwise_and(item, Q - 1)
      rv = roisv[n]
      b = rv[0].astype(jnp.int32)
      rs = _round_half_even_v(rv * SCALE)
      rsw, rsh, rew, reh = rs[1], rs[2], rs[3], rs[4]
      roi_w = jnp.maximum(rew - rsw + 1, 1)
      roi_h = jnp.maximum(reh - rsh + 1, 1)
      bin_h = div7v[roi_h][0]
      bin_w = div7v[roi_w][0]
      hs_v = jnp.clip(_floor_v(iot_f * bin_h) + rsh, 0, H)
      he_v = jnp.clip(_ceil_v((iot_f + 1.0) * bin_h) + rsh, 0, H)
      ws_v = jnp.clip(_floor_v(iot_f * bin_w) + rsw, 0, W)
      we_v = jnp.clip(_ceil_v((iot_f + 1.0) * bin_w) + rsw, 0, W)
      h0 = hs_v[0]
      nh = he_v[POOLED - 1] - h0
      s_sel = jnp.int32(NH_BUCKETS[-1])
      for s in reversed(NH_BUCKETS[:-1]):
        s_sel = jnp.where(nh <= s, jnp.int32(s), s_sel)
      h0c = jnp.minimum(h0, H - s_sel)
      return dict(n=n, q=q, b=b, nh=nh, h0c=h0c,
                  hs=hs_v, he=he_v, ws=ws_v, we=we_v)

    def issue_in(prm, slab, sem):
      prev = 0
      for s in NH_BUCKETS:
        cond = (prm["nh"] > prev) & (prm["nh"] <= s) if prev else (prm["nh"] <= s)

        @pl.when(cond)
        def _(s=s):
          pltpu.async_copy(
              featq_hbm.at[prm["q"], prm["b"],
                           pl.ds(jnp.minimum(prm["h0c"], H - s), s)],
              slab.at[pl.ds(0, s)], sem)

        prev = s

    def drain_in(prm, slab, sem):
      prev = 0
      for s in NH_BUCKETS:
        cond = (prm["nh"] > prev) & (prm["nh"] <= s) if prev else (prm["nh"] <= s)

        @pl.when(cond)
        def _(s=s):
          pltpu.make_async_copy(
              featq_hbm.at[0, 0, pl.ds(0, s)], slab.at[pl.ds(0, s)], sem
          ).wait()

        prev = s

    def drain_out(ob, sem):
      pltpu.make_async_copy(out_hbm.at[0, pl.ds(0, CCHUNK)], ob, sem).wait()

    def compute(prm, slab, ob, semo, first):
      @pl.when(jnp.logical_not(first))
      def _():
        drain_out(ob, semo)

      h0c = prm["h0c"]
      neg = jnp.full((16,), -jnp.inf, jnp.float32)
      wlens = jnp.where(cidx < POOLED, prm["we"] - prm["ws"], 0)
      wcap = lax.reduce_max(wlens, (0,))
      ws_s = [prm["ws"][pw] for pw in range(POOLED)]
      wl_s = [jnp.maximum(prm["we"][pw] - 1, 0) for pw in range(POOLED)]
      for ph in range(POOLED):
        hs, he = prm["hs"][ph], prm["he"][ph]
        phv = jnp.broadcast_to(jnp.int32(ph), (16,))

        def hbody(h, accs, _ph=ph):
          row = h - h0c

          def jbody(j, accs2):
            out = []
            for pw in range(POOLED):
              a0, a1 = accs2[2 * pw], accs2[2 * pw + 1]
              col = jnp.minimum(ws_s[pw] + j, wl_s[pw]) * CCHUNK
              v0 = slab[row, pl.ds(col, 16)]
              v1 = slab[row, pl.ds(col + 16, 16)]
              out.append(jnp.maximum(a0, v0))
              out.append(jnp.maximum(a1, v1))
            return tuple(out)

          return lax.fori_loop(0, wcap, jbody, accs)

        accs = lax.fori_loop(hs, he, hbody, (neg,) * (2 * POOLED))
        for pw in range(POOLED):
          ws, we = prm["ws"][pw], prm["we"][pw]
          emptyv = jnp.broadcast_to((he <= hs) | (we <= ws), (16,))
          r0 = jnp.where(emptyv, 0.0, accs[2 * pw])
          r1 = jnp.where(emptyv, 0.0, accs[2 * pw + 1])
          pwv = jnp.broadcast_to(jnp.int32(pw), (16,))
          plsc.store_scatter(ob, [cidx, phv, pwv], r0)
          plsc.store_scatter(ob, [cidx + 16, phv, pwv], r1)
      pltpu.async_copy(
          ob, out_hbm.at[prm["n"], pl.ds(prm["q"] * CCHUNK, CCHUNK)], semo)

    p0 = params(0)
    issue_in(p0, slab0, semA)

    @pl.loop(0, ipw // 2)
    def _pair(p):
      k0 = 2 * p
      prm1 = params(k0 + 1)
      issue_in(prm1, slab1, semB)
      prm0 = params(k0)
      drain_in(prm0, slab0, semA)
      compute(prm0, slab0, ob0, semO0, first=(k0 == 0))
      prm2 = params(jnp.minimum(k0 + 2, ipw - 1))
      issue_in(prm2, slab0, semA)
      drain_in(prm1, slab1, semB)
      compute(prm1, slab1, ob1, semO1, first=(k0 == 0))

    plast = params(ipw - 1)
    drain_in(plast, slab0, semA)
    drain_out(ob0, semO0)
    drain_out(ob1, semO1)

  mesh = plsc.VectorSubcoreMesh(core_axis_name="c", subcore_axis_name="s")
  return pl.kernel(
      body,
      out_type=jax.ShapeDtypeStruct((N, C, POOLED, POOLED), jnp.float32),
      mesh=mesh,
      compiler_params=pltpu.CompilerParams(
          use_tc_tiling_on_sc=False, needs_layout_passes=False
      ),
      scratch_types=[
          pltpu.VMEM((N, 16), jnp.float32),
          pltpu.VMEM((64, 16), jnp.float32),
          pltpu.VMEM((H, W * CCHUNK), jnp.float32),
          pltpu.VMEM((H, W * CCHUNK), jnp.float32),
          pltpu.VMEM((CCHUNK, POOLED, POOLED), jnp.float32),
          pltpu.VMEM((CCHUNK, POOLED, POOLED), jnp.float32),
          pltpu.SemaphoreType.DMA,
          pltpu.SemaphoreType.DMA,
          pltpu.SemaphoreType.DMA,
          pltpu.SemaphoreType.DMA,
      ],
  )


@jax.jit
def kernel(features, rois):
  B, C, H, W = features.shape
  N = rois.shape[0]
  Q = C // CCHUNK
  featq = (
      features.reshape(B, Q, CCHUNK, H, W)
      .transpose(1, 0, 3, 4, 2)
      .reshape(Q, B, H, W * CCHUNK)
  )
  roisp = jnp.pad(rois, ((0, 0), (0, 16 - rois.shape[1])))
  div7 = jnp.asarray(
      np.broadcast_to(
          (
              np.arange(64, dtype=np.float32)
              * (np.float32(1.0) / np.float32(POOLED))
          )[:, None],
          (64, 16),
      )
  )
  return _make_sc_call(N, C, B, H, W)(featq, roisp, div7)
